# baseline (speedup 1.0000x reference)
import jax
import jax.numpy as jnp
from jax import lax
from jax.experimental import pallas as pl
from jax.experimental.pallas import tpu as pltpu
from jax.experimental.pallas import tpu_sc as plsc

_HIDDEN = 2048
_HEADS = 16
_BATCH = 4
_SEQ = 8192
_NW = 32
_WPB = _NW // _BATCH
_SC_ROWS = 2048
_W = _SC_ROWS // _WPB
_SC_CHUNK = 16
_TC_ROWS = _SEQ - _SC_ROWS
_CROWS = 512
_NBUF = 4
_NEG = -jnp.finfo(jnp.float32).max


def _sc_pool_body(x_hbm, parts_hbm, buf, acc_ref, sem):
    wid = lax.axis_index("s") * 2 + lax.axis_index("c")
    b = wid // _WPB
    piece = lax.rem(wid, _WPB)
    row0 = b * _SEQ + _TC_ROWS + piece * _W
    nchunk = _W // _SC_CHUNK

    def copy_in(k, slot):
        return pltpu.make_async_copy(
            x_hbm.at[pl.ds(row0 + k * _SC_CHUNK, _SC_CHUNK), :],
            buf.at[slot], sem.at[slot])

    copy_in(0, 0).start()
    copy_in(1, 1).start()

    zeros = jnp.zeros((16,), jnp.float32)
    for c in range(_HIDDEN // 16):
        acc_ref[pl.ds(c * 16, 16)] = zeros

    def chunk_pair(k2, carry):
        for slot in range(2):
            k = k2 * 2 + slot
            copy_in(k, slot).wait()

            def row_body(r, rc):
                for c in range(_HIDDEN // 16):
                    v = buf[slot, r, pl.ds(c * 16, 16)]
                    plsc.addupdate(acc_ref.at[pl.ds(c * 16, 16)], v)
                return rc

            lax.fori_loop(0, _SC_CHUNK, row_body, 0)
            nxt = k + 2

            @pl.when(nxt < nchunk)
            def _():
                copy_in(nxt, slot).start()
        return carry

    lax.fori_loop(0, nchunk // 2, chunk_pair, 0)
    pltpu.sync_copy(acc_ref, parts_hbm.at[wid])


def _sc_pool(x_flat):
    mesh = plsc.VectorSubcoreMesh(
        core_axis_name="c", subcore_axis_name="s",
        num_cores=2, num_subcores=16)
    fn = pl.kernel(
        _sc_pool_body,
        out_type=jax.ShapeDtypeStruct((_NW, _HIDDEN), jnp.float32),
        mesh=mesh,
        scratch_types=[
            pltpu.VMEM((2, _SC_CHUNK, _HIDDEN), jnp.float32),
            pltpu.VMEM((_HIDDEN,), jnp.float32),
            pltpu.SemaphoreType.DMA((2,)),
        ],
    )
    return fn(x_flat)


def _tc_pool_body(x_hbm, acc_out_ref, buf, sem):
    n = (_BATCH * _TC_ROWS) // _CROWS
    per_b = _TC_ROWS // _CROWS

    def copy_in(i, s):
        b = i // per_b
        j = lax.rem(i, per_b)
        return pltpu.make_async_copy(
            x_hbm.at[pl.ds(b * _SEQ + j * _CROWS, _CROWS), :],
            buf.at[s], sem.at[s])

    for s in range(_NBUF):
        copy_in(s, s).start()
    acc_out_ref[...] = jnp.zeros_like(acc_out_ref)

    def outer(o, carry):
        for s in range(_NBUF):
            i = o * _NBUF + s
            copy_in(i, s).wait()
            partial = jnp.sum(buf[s], axis=0, keepdims=True)
            b = i // per_b
            acc_out_ref[pl.ds(b, 1), :] += partial
            nxt = i + _NBUF

            @pl.when(nxt < n)
            def _():
                copy_in(nxt, s).start()
        return carry

    lax.fori_loop(0, n // _NBUF, outer, 0)


def _tc_pool(x_flat):
    return pl.pallas_call(
        _tc_pool_body,
        in_specs=[pl.BlockSpec(memory_space=pl.ANY)],
        out_specs=pl.BlockSpec((_BATCH, _HIDDEN), lambda: (0, 0)),
        out_shape=jax.ShapeDtypeStruct((_BATCH, _HIDDEN), jnp.float32),
        scratch_shapes=[
            pltpu.VMEM((_NBUF, _CROWS, _HIDDEN), jnp.float32),
            pltpu.SemaphoreType.DMA((_NBUF,)),
        ],
    )(x_flat)


def _combine_body(tc_acc_ref, parts_ref, sim_ref, gates_ref,
                  probs_ref, pre_ref, mask_ref):
    parts = parts_ref[...].reshape(_BATCH, _WPB, _HIDDEN)
    pooled = (tc_acc_ref[...] + jnp.sum(parts, axis=1)) * (1.0 / _SEQ)
    pnorm = jnp.sqrt(jnp.sum(pooled * pooled, axis=-1, keepdims=True))
    pooled_n = pooled / jnp.maximum(pnorm, 1e-12)
    sim = sim_ref[...]
    snorm = jnp.sqrt(jnp.sum(sim * sim, axis=0, keepdims=True))
    sim_n = sim / jnp.maximum(snorm, 1e-12)
    logits = jnp.dot(pooled_n, sim_n, preferred_element_type=jnp.float32)

    pre = logits - jax.nn.sigmoid(gates_ref[...])
    gated = jnp.maximum(pre, 0.0)
    ind = jnp.where(pre > 0.0, 1.0, 0.0)
    inactive = jnp.max(pre, axis=-1, keepdims=True) <= 0.0
    ci = lax.broadcasted_iota(jnp.int32, (_BATCH, _HEADS), 1)
    rank = jnp.zeros((_BATCH, _HEADS), jnp.float32)
    for j in range(_HEADS):
        lj = logits[:, j:j + 1]
        beats = (lj > logits) | ((lj == logits) & (ci > j))
        rank = rank + jnp.where(beats, 1.0, 0.0)
    fb = jnp.where(rank < float(_HEADS // 2), 1.0, 0.0)
    mask = jnp.where(inactive, fb, ind)
    gm = jnp.where(mask > 0.0, gated, _NEG)
    m = jnp.max(gm, axis=-1, keepdims=True)
    e = jnp.exp(gm - m)
    probs_ref[...] = e / jnp.sum(e, axis=-1, keepdims=True)
    pre_ref[...] = pre
    mask_ref[...] = mask


def _combine(tc_acc, sc_parts, sim_matrix, gates2d):
    out = jax.ShapeDtypeStruct((_BATCH, _HEADS), jnp.float32)
    return pl.pallas_call(
        _combine_body,
        in_specs=[
            pl.BlockSpec((_BATCH, _HIDDEN), lambda: (0, 0)),
            pl.BlockSpec((_NW, _HIDDEN), lambda: (0, 0)),
            pl.BlockSpec((_HIDDEN, _HEADS), lambda: (0, 0)),
            pl.BlockSpec((1, _HEADS), lambda: (0, 0)),
        ],
        out_specs=[
            pl.BlockSpec((_BATCH, _HEADS), lambda: (0, 0)),
            pl.BlockSpec((_BATCH, _HEADS), lambda: (0, 0)),
            pl.BlockSpec((_BATCH, _HEADS), lambda: (0, 0)),
        ],
        out_shape=[out, out, out],
    )(tc_acc, sc_parts, sim_matrix, gates2d)


def kernel(x, sim_matrix, gates):
    x_flat = x.reshape(_BATCH * _SEQ, _HIDDEN)
    sc_parts = _sc_pool(x_flat)
    tc_acc = _tc_pool(x_flat)
    probs, pre, mask = _combine(
        tc_acc, sc_parts, sim_matrix, gates.reshape(1, _HEADS))
    return (probs, pre, mask)


# speedup vs baseline: 1.4197x; 1.4197x over previous
import jax
import jax.numpy as jnp
from jax import lax
from jax.experimental import pallas as pl
from jax.experimental.pallas import tpu as pltpu
from jax.experimental.pallas import tpu_sc as plsc

_HIDDEN = 2048
_HEADS = 16
_BATCH = 4
_SEQ = 8192
_NW = 32
_WPB = _NW // _BATCH
_SC_ROWS = 2048
_W = _SC_ROWS // _WPB
_SC_CHUNK = 16
_TC_ROWS = _SEQ - _SC_ROWS
_CROWS = 512
_NBUF = 4
_NEG = -jnp.finfo(jnp.float32).max


def _sc_pool_body(x_hbm, parts_hbm, buf, acc_ref, sem):
    wid = lax.axis_index("s") * 2 + lax.axis_index("c")
    b = wid // _WPB
    piece = lax.rem(wid, _WPB)
    row0 = b * _SEQ + _TC_ROWS + piece * _W
    nchunk = _W // _SC_CHUNK

    def copy_in(k, slot):
        return pltpu.make_async_copy(
            x_hbm.at[pl.ds(row0 + k * _SC_CHUNK, _SC_CHUNK), :],
            buf.at[slot], sem.at[slot])

    copy_in(0, 0).start()
    copy_in(1, 1).start()

    zeros = jnp.zeros((16,), jnp.float32)
    for c in range(_HIDDEN // 16):
        acc_ref[pl.ds(c * 16, 16)] = zeros

    ngroup = 4
    gcols = _HIDDEN // ngroup
    nslice = gcols // 16

    def chunk_pair(k2, carry):
        for slot in range(2):
            k = k2 * 2 + slot
            copy_in(k, slot).wait()
            for g in range(ngroup):
                def row_body(r, accs, _g=g):
                    return tuple(
                        accs[c] + buf[slot, r, pl.ds(_g * gcols + c * 16, 16)]
                        for c in range(nslice))

                zeros = tuple(
                    jnp.zeros((16,), jnp.float32) for _ in range(nslice))
                accs = lax.fori_loop(0, _SC_CHUNK, row_body, zeros)
                for c in range(nslice):
                    plsc.addupdate(
                        acc_ref.at[pl.ds(g * gcols + c * 16, 16)], accs[c])
            nxt = k + 2

            @pl.when(nxt < nchunk)
            def _():
                copy_in(nxt, slot).start()
        return carry

    lax.fori_loop(0, nchunk // 2, chunk_pair, 0)
    pltpu.sync_copy(acc_ref, parts_hbm.at[wid])


def _sc_pool(x_flat):
    mesh = plsc.VectorSubcoreMesh(
        core_axis_name="c", subcore_axis_name="s",
        num_cores=2, num_subcores=16)
    fn = pl.kernel(
        _sc_pool_body,
        out_type=jax.ShapeDtypeStruct((_NW, _HIDDEN), jnp.float32),
        mesh=mesh,
        scratch_types=[
            pltpu.VMEM((2, _SC_CHUNK, _HIDDEN), jnp.float32),
            pltpu.VMEM((_HIDDEN,), jnp.float32),
            pltpu.SemaphoreType.DMA((2,)),
        ],
    )
    return fn(x_flat)


def _tc_pool_body(x_hbm, acc_out_ref, buf, sem):
    n = (_BATCH * _TC_ROWS) // _CROWS
    per_b = _TC_ROWS // _CROWS

    def copy_in(i, s):
        b = i // per_b
        j = lax.rem(i, per_b)
        return pltpu.make_async_copy(
            x_hbm.at[pl.ds(b * _SEQ + j * _CROWS, _CROWS), :],
            buf.at[s], sem.at[s])

    for s in range(_NBUF):
        copy_in(s, s).start()
    acc_out_ref[...] = jnp.zeros_like(acc_out_ref)

    def outer(o, carry):
        for s in range(_NBUF):
            i = o * _NBUF + s
            copy_in(i, s).wait()
            partial = jnp.sum(buf[s], axis=0, keepdims=True)
            b = i // per_b
            acc_out_ref[pl.ds(b, 1), :] += partial
            nxt = i + _NBUF

            @pl.when(nxt < n)
            def _():
                copy_in(nxt, s).start()
        return carry

    lax.fori_loop(0, n // _NBUF, outer, 0)


def _tc_pool(x_flat):
    return pl.pallas_call(
        _tc_pool_body,
        in_specs=[pl.BlockSpec(memory_space=pl.ANY)],
        out_specs=pl.BlockSpec((_BATCH, _HIDDEN), lambda: (0, 0)),
        out_shape=jax.ShapeDtypeStruct((_BATCH, _HIDDEN), jnp.float32),
        scratch_shapes=[
            pltpu.VMEM((_NBUF, _CROWS, _HIDDEN), jnp.float32),
            pltpu.SemaphoreType.DMA((_NBUF,)),
        ],
    )(x_flat)


def _combine_body(tc_acc_ref, parts_ref, sim_ref, gates_ref,
                  probs_ref, pre_ref, mask_ref):
    parts = parts_ref[...].reshape(_BATCH, _WPB, _HIDDEN)
    pooled = (tc_acc_ref[...] + jnp.sum(parts, axis=1)) * (1.0 / _SEQ)
    pnorm = jnp.sqrt(jnp.sum(pooled * pooled, axis=-1, keepdims=True))
    pooled_n = pooled / jnp.maximum(pnorm, 1e-12)
    sim = sim_ref[...]
    snorm = jnp.sqrt(jnp.sum(sim * sim, axis=0, keepdims=True))
    sim_n = sim / jnp.maximum(snorm, 1e-12)
    logits = jnp.dot(pooled_n, sim_n, preferred_element_type=jnp.float32)

    pre = logits - jax.nn.sigmoid(gates_ref[...])
    gated = jnp.maximum(pre, 0.0)
    ind = jnp.where(pre > 0.0, 1.0, 0.0)
    inactive = jnp.max(pre, axis=-1, keepdims=True) <= 0.0
    ci = lax.broadcasted_iota(jnp.int32, (_BATCH, _HEADS), 1)
    rank = jnp.zeros((_BATCH, _HEADS), jnp.float32)
    for j in range(_HEADS):
        lj = logits[:, j:j + 1]
        beats = (lj > logits) | ((lj == logits) & (ci > j))
        rank = rank + jnp.where(beats, 1.0, 0.0)
    fb = jnp.where(rank < float(_HEADS // 2), 1.0, 0.0)
    mask = jnp.where(inactive, fb, ind)
    gm = jnp.where(mask > 0.0, gated, _NEG)
    m = jnp.max(gm, axis=-1, keepdims=True)
    e = jnp.exp(gm - m)
    probs_ref[...] = e / jnp.sum(e, axis=-1, keepdims=True)
    pre_ref[...] = pre
    mask_ref[...] = mask


def _combine(tc_acc, sc_parts, sim_matrix, gates2d):
    out = jax.ShapeDtypeStruct((_BATCH, _HEADS), jnp.float32)
    return pl.pallas_call(
        _combine_body,
        in_specs=[
            pl.BlockSpec((_BATCH, _HIDDEN), lambda: (0, 0)),
            pl.BlockSpec((_NW, _HIDDEN), lambda: (0, 0)),
            pl.BlockSpec((_HIDDEN, _HEADS), lambda: (0, 0)),
            pl.BlockSpec((1, _HEADS), lambda: (0, 0)),
        ],
        out_specs=[
            pl.BlockSpec((_BATCH, _HEADS), lambda: (0, 0)),
            pl.BlockSpec((_BATCH, _HEADS), lambda: (0, 0)),
            pl.BlockSpec((_BATCH, _HEADS), lambda: (0, 0)),
        ],
        out_shape=[out, out, out],
    )(tc_acc, sc_parts, sim_matrix, gates2d)


def kernel(x, sim_matrix, gates):
    x_flat = x.reshape(_BATCH * _SEQ, _HIDDEN)
    sc_parts = _sc_pool(x_flat)
    tc_acc = _tc_pool(x_flat)
    probs, pre, mask = _combine(
        tc_acc, sc_parts, sim_matrix, gates.reshape(1, _HEADS))
    return (probs, pre, mask)


# speedup vs baseline: 1.4389x; 1.0135x over previous
import jax
import jax.numpy as jnp
from jax import lax
from jax.experimental import pallas as pl
from jax.experimental.pallas import tpu as pltpu
from jax.experimental.pallas import tpu_sc as plsc

_HIDDEN = 2048
_HEADS = 16
_BATCH = 4
_SEQ = 8192
_NW = 32
_WPB = _NW // _BATCH
_SC_ROWS = 1024
_W = _SC_ROWS // _WPB
_SC_CHUNK = 16
_TC_ROWS = _SEQ - _SC_ROWS
_CROWS = 512
_NBUF = 4
_NEG = -jnp.finfo(jnp.float32).max


def _sc_pool_body(x_hbm, parts_hbm, buf, acc_ref, sem):
    wid = lax.axis_index("s") * 2 + lax.axis_index("c")
    b = wid // _WPB
    piece = lax.rem(wid, _WPB)
    row0 = b * _SEQ + _TC_ROWS + piece * _W
    nchunk = _W // _SC_CHUNK

    def copy_in(k, slot):
        return pltpu.make_async_copy(
            x_hbm.at[pl.ds(row0 + k * _SC_CHUNK, _SC_CHUNK), :],
            buf.at[slot], sem.at[slot])

    copy_in(0, 0).start()
    copy_in(1, 1).start()

    zeros = jnp.zeros((16,), jnp.float32)
    for c in range(_HIDDEN // 16):
        acc_ref[pl.ds(c * 16, 16)] = zeros

    ngroup = 4
    gcols = _HIDDEN // ngroup
    nslice = gcols // 16

    def chunk_pair(k2, carry):
        for slot in range(2):
            k = k2 * 2 + slot
            copy_in(k, slot).wait()
            for g in range(ngroup):
                def row_body(r, accs, _g=g):
                    return tuple(
                        accs[c] + buf[slot, r, pl.ds(_g * gcols + c * 16, 16)]
                        for c in range(nslice))

                zeros = tuple(
                    jnp.zeros((16,), jnp.float32) for _ in range(nslice))
                accs = lax.fori_loop(0, _SC_CHUNK, row_body, zeros)
                for c in range(nslice):
                    plsc.addupdate(
                        acc_ref.at[pl.ds(g * gcols + c * 16, 16)], accs[c])
            nxt = k + 2

            @pl.when(nxt < nchunk)
            def _():
                copy_in(nxt, slot).start()
        return carry

    lax.fori_loop(0, nchunk // 2, chunk_pair, 0)
    pltpu.sync_copy(acc_ref, parts_hbm.at[wid])


def _sc_pool(x_flat):
    mesh = plsc.VectorSubcoreMesh(
        core_axis_name="c", subcore_axis_name="s",
        num_cores=2, num_subcores=16)
    fn = pl.kernel(
        _sc_pool_body,
        out_type=jax.ShapeDtypeStruct((_NW, _HIDDEN), jnp.float32),
        mesh=mesh,
        scratch_types=[
            pltpu.VMEM((2, _SC_CHUNK, _HIDDEN), jnp.float32),
            pltpu.VMEM((_HIDDEN,), jnp.float32),
            pltpu.SemaphoreType.DMA((2,)),
        ],
    )
    return fn(x_flat)


def _tc_pool_body(x_hbm, acc_out_ref, buf, sem):
    n = (_BATCH * _TC_ROWS) // _CROWS
    per_b = _TC_ROWS // _CROWS

    def copy_in(i, s):
        b = i // per_b
        j = lax.rem(i, per_b)
        return pltpu.make_async_copy(
            x_hbm.at[pl.ds(b * _SEQ + j * _CROWS, _CROWS), :],
            buf.at[s], sem.at[s])

    for s in range(_NBUF):
        copy_in(s, s).start()
    acc_out_ref[...] = jnp.zeros_like(acc_out_ref)

    def outer(o, carry):
        for s in range(_NBUF):
            i = o * _NBUF + s
            copy_in(i, s).wait()
            partial = jnp.sum(buf[s], axis=0, keepdims=True)
            b = i // per_b
            acc_out_ref[pl.ds(b, 1), :] += partial
            nxt = i + _NBUF

            @pl.when(nxt < n)
            def _():
                copy_in(nxt, s).start()
        return carry

    lax.fori_loop(0, n // _NBUF, outer, 0)


def _tc_pool(x_flat):
    return pl.pallas_call(
        _tc_pool_body,
        in_specs=[pl.BlockSpec(memory_space=pl.ANY)],
        out_specs=pl.BlockSpec((_BATCH, _HIDDEN), lambda: (0, 0)),
        out_shape=jax.ShapeDtypeStruct((_BATCH, _HIDDEN), jnp.float32),
        scratch_shapes=[
            pltpu.VMEM((_NBUF, _CROWS, _HIDDEN), jnp.float32),
            pltpu.SemaphoreType.DMA((_NBUF,)),
        ],
    )(x_flat)


def _combine_body(tc_acc_ref, parts_ref, sim_ref, gates_ref,
                  probs_ref, pre_ref, mask_ref):
    parts = parts_ref[...].reshape(_BATCH, _WPB, _HIDDEN)
    pooled = (tc_acc_ref[...] + jnp.sum(parts, axis=1)) * (1.0 / _SEQ)
    pnorm = jnp.sqrt(jnp.sum(pooled * pooled, axis=-1, keepdims=True))
    pooled_n = pooled / jnp.maximum(pnorm, 1e-12)
    sim = sim_ref[...]
    snorm = jnp.sqrt(jnp.sum(sim * sim, axis=0, keepdims=True))
    sim_n = sim / jnp.maximum(snorm, 1e-12)
    logits = jnp.dot(pooled_n, sim_n, preferred_element_type=jnp.float32)

    pre = logits - jax.nn.sigmoid(gates_ref[...])
    gated = jnp.maximum(pre, 0.0)
    ind = jnp.where(pre > 0.0, 1.0, 0.0)
    inactive = jnp.max(pre, axis=-1, keepdims=True) <= 0.0
    ci = lax.broadcasted_iota(jnp.int32, (_BATCH, _HEADS), 1)
    rank = jnp.zeros((_BATCH, _HEADS), jnp.float32)
    for j in range(_HEADS):
        lj = logits[:, j:j + 1]
        beats = (lj > logits) | ((lj == logits) & (ci > j))
        rank = rank + jnp.where(beats, 1.0, 0.0)
    fb = jnp.where(rank < float(_HEADS // 2), 1.0, 0.0)
    mask = jnp.where(inactive, fb, ind)
    gm = jnp.where(mask > 0.0, gated, _NEG)
    m = jnp.max(gm, axis=-1, keepdims=True)
    e = jnp.exp(gm - m)
    probs_ref[...] = e / jnp.sum(e, axis=-1, keepdims=True)
    pre_ref[...] = pre
    mask_ref[...] = mask


def _combine(tc_acc, sc_parts, sim_matrix, gates2d):
    out = jax.ShapeDtypeStruct((_BATCH, _HEADS), jnp.float32)
    return pl.pallas_call(
        _combine_body,
        in_specs=[
            pl.BlockSpec((_BATCH, _HIDDEN), lambda: (0, 0)),
            pl.BlockSpec((_NW, _HIDDEN), lambda: (0, 0)),
            pl.BlockSpec((_HIDDEN, _HEADS), lambda: (0, 0)),
            pl.BlockSpec((1, _HEADS), lambda: (0, 0)),
        ],
        out_specs=[
            pl.BlockSpec((_BATCH, _HEADS), lambda: (0, 0)),
            pl.BlockSpec((_BATCH, _HEADS), lambda: (0, 0)),
            pl.BlockSpec((_BATCH, _HEADS), lambda: (0, 0)),
        ],
        out_shape=[out, out, out],
    )(tc_acc, sc_parts, sim_matrix, gates2d)


def kernel(x, sim_matrix, gates):
    x_flat = x.reshape(_BATCH * _SEQ, _HIDDEN)
    sc_parts = _sc_pool(x_flat)
    tc_acc = _tc_pool(x_flat)
    probs, pre, mask = _combine(
        tc_acc, sc_parts, sim_matrix, gates.reshape(1, _HEADS))
    return (probs, pre, mask)


# speedup vs baseline: 1.7241x; 1.1982x over previous
import jax
import jax.numpy as jnp
from jax import lax
from jax.experimental import pallas as pl
from jax.experimental.pallas import tpu as pltpu

_HIDDEN = 2048
_HEADS = 16
_BATCH = 4
_SEQ = 8192
_CROWS = 256
_NBUF = 8
_NEG = -jnp.finfo(jnp.float32).max


def _gate_from_logits(logits, gates_row):
    pre = logits - jax.nn.sigmoid(gates_row)
    gated = jnp.maximum(pre, 0.0)
    ind = jnp.where(pre > 0.0, 1.0, 0.0)
    inactive = jnp.max(pre, axis=-1, keepdims=True) <= 0.0
    ci = lax.broadcasted_iota(jnp.int32, (_BATCH, _HEADS), 1)
    rank = jnp.zeros((_BATCH, _HEADS), jnp.float32)
    for j in range(_HEADS):
        lj = logits[:, j:j + 1]
        beats = (lj > logits) | ((lj == logits) & (ci > j))
        rank = rank + jnp.where(beats, 1.0, 0.0)
    fb = jnp.where(rank < float(_HEADS // 2), 1.0, 0.0)
    mask = jnp.where(inactive, fb, ind)
    gm = jnp.where(mask > 0.0, gated, _NEG)
    m = jnp.max(gm, axis=-1, keepdims=True)
    e = jnp.exp(gm - m)
    probs = e / jnp.sum(e, axis=-1, keepdims=True)
    return probs, pre, mask


def _body(x_hbm, sim_ref, gates_ref, probs_ref, pre_ref, mask_ref,
          buf, sem):
    n = (_BATCH * _SEQ) // _CROWS
    per_b = _SEQ // _CROWS
    rows = lax.broadcasted_iota(jnp.int32, (_BATCH, 1), 0)

    def copy_in(i, s):
        return pltpu.make_async_copy(
            x_hbm.at[pl.ds(i * _CROWS, _CROWS), :], buf.at[s], sem.at[s])

    for s in range(_NBUF):
        copy_in(s, s).start()

    def outer(o, acc):
        for s in range(_NBUF):
            i = o * _NBUF + s
            copy_in(i, s).wait()
            partial = jnp.sum(buf[s], axis=0, keepdims=True)
            b = i // per_b
            acc = acc + jnp.where(rows == b, partial, 0.0)
            nxt = i + _NBUF

            @pl.when(nxt < n)
            def _():
                copy_in(nxt, s).start()
        return acc

    acc = lax.fori_loop(
        0, n // _NBUF, outer, jnp.zeros((_BATCH, _HIDDEN), jnp.float32))

    pooled = acc * (1.0 / _SEQ)
    pnorm = jnp.sqrt(jnp.sum(pooled * pooled, axis=-1, keepdims=True))
    pooled_n = pooled / jnp.maximum(pnorm, 1e-12)
    sim = sim_ref[...]
    snorm = jnp.sqrt(jnp.sum(sim * sim, axis=0, keepdims=True))
    sim_n = sim / jnp.maximum(snorm, 1e-12)
    logits = jnp.dot(pooled_n, sim_n, preferred_element_type=jnp.float32)
    probs, pre, mask = _gate_from_logits(logits, gates_ref[...])
    probs_ref[...] = probs
    pre_ref[...] = pre
    mask_ref[...] = mask


def kernel(x, sim_matrix, gates):
    x_flat = x.reshape(_BATCH * _SEQ, _HIDDEN)
    out = jax.ShapeDtypeStruct((_BATCH, _HEADS), jnp.float32)
    probs, pre, mask = pl.pallas_call(
        _body,
        in_specs=[
            pl.BlockSpec(memory_space=pl.ANY),
            pl.BlockSpec((_HIDDEN, _HEADS), lambda: (0, 0)),
            pl.BlockSpec((1, _HEADS), lambda: (0, 0)),
        ],
        out_specs=[
            pl.BlockSpec((_BATCH, _HEADS), lambda: (0, 0)),
            pl.BlockSpec((_BATCH, _HEADS), lambda: (0, 0)),
            pl.BlockSpec((_BATCH, _HEADS), lambda: (0, 0)),
        ],
        out_shape=[out, out, out],
        scratch_shapes=[
            pltpu.VMEM((_NBUF, _CROWS, _HIDDEN), jnp.float32),
            pltpu.SemaphoreType.DMA((_NBUF,)),
        ],
    )(x_flat, sim_matrix, gates.reshape(1, _HEADS))
    return (probs, pre, mask)
